# Initial kernel scaffold; baseline (speedup 1.0000x reference)
import jax
import jax.numpy as jnp
from jax.experimental import pallas as pl


def kernel(features, edge_index, W1, b1, W2, b2, W3, b3):
    raise NotImplementedError("write your pallas kernel here")


# speedup vs baseline: 5.3035x; 5.3035x over previous
import functools

import jax
import jax.numpy as jnp
from jax import lax
from jax.experimental import pallas as pl
from jax.experimental.pallas import tpu as pltpu
from jax.experimental.pallas import tpu_sc as plsc

N = 10000
E = 160000
D_IN = 256
D_HID = 256
D_OUT = 64

NC = 2
NS = 16
WIN = 128
WPT = 80
E_PAD = NS * WPT * WIN
N_ACC = 10112
RPT = N_ACC // NS
ZPT = N_ACC // NS

_MESH = plsc.VectorSubcoreMesh(core_axis_name="c", subcore_axis_name="s")


def _zero_rows(buf, nrows, ncols):
    def row(i, _):
        for j in range(ncols // 16):
            buf[i, pl.ds(j * 16, 16)] = jnp.zeros((16,), jnp.float32)
        return 0
    lax.fori_loop(0, nrows, row, 0)


def _zero_acc(z_buf, acc, s):
    for k in range(ZPT // WIN + 1):
        st = s * ZPT + min(k * WIN, ZPT - WIN)
        pltpu.sync_copy(z_buf, acc.at[pl.ds(st, WIN)])


@functools.partial(
    pl.kernel,
    mesh=_MESH,
    out_type=[
        jax.ShapeDtypeStruct((N_ACC, 128), jnp.float32),
        jax.ShapeDtypeStruct((N_ACC, 128), jnp.float32),
    ],
    scratch_types=[
        pltpu.VMEM((WPT, WIN), jnp.int32),
        pltpu.VMEM((WIN, 128), jnp.float32),
        pltpu.VMEM_SHARED((N_ACC, 128), jnp.float32),
    ],
)
def _deg_kernel(dst_hbm, outa_hbm, outb_hbm, dst_v, ones_v, acc):
    c = lax.axis_index("c")
    s = lax.axis_index("s")
    _zero_rows(ones_v, WIN, 128)
    _zero_acc(ones_v, acc, s)
    def row(i, _):
        ones_v[i, pl.ds(0, 16)] = jnp.ones((16,), jnp.float32)
        return 0
    lax.fori_loop(0, WIN, row, 0)
    pltpu.sync_copy(dst_hbm.at[s], dst_v)
    plsc.subcore_barrier()

    half = WPT // NC
    def win(w, _):
        pltpu.sync_copy(ones_v, acc.at[dst_v.at[w]], add=True)
        return 0
    lax.fori_loop(c * half, (c + 1) * half, win, 0)
    plsc.subcore_barrier()

    @pl.when(c == 0)
    def _():
        pltpu.sync_copy(acc.at[pl.ds(s * RPT, RPT)], outa_hbm.at[pl.ds(s * RPT, RPT)])

    @pl.when(c == 1)
    def _():
        pltpu.sync_copy(acc.at[pl.ds(s * RPT, RPT)], outb_hbm.at[pl.ds(s * RPT, RPT)])


@functools.partial(
    pl.kernel,
    mesh=_MESH,
    out_type=[
        jax.ShapeDtypeStruct((N_ACC, 128), jnp.float32),
        jax.ShapeDtypeStruct((N_ACC, 128), jnp.float32),
    ],
    scratch_types=[
        pltpu.VMEM((WPT, WIN), jnp.int32),
        pltpu.VMEM((WPT, WIN), jnp.int32),
        pltpu.VMEM((WIN, 128), jnp.float32),
        pltpu.VMEM_SHARED((N_ACC, 128), jnp.float32),
        pltpu.SemaphoreType.DMA,
    ],
)
def _scatter128_kernel(ta_hbm, tb_hbm, src_hbm, dst_hbm, outa_hbm, outb_hbm,
                       src_v, dst_v, rows_v, acc, sem):
    c = lax.axis_index("c")
    s = lax.axis_index("s")
    _zero_rows(rows_v, WIN, 128)
    _zero_acc(rows_v, acc, s)
    pltpu.sync_copy(src_hbm.at[s], src_v)
    pltpu.sync_copy(dst_hbm.at[s], dst_v)
    plsc.subcore_barrier()

    def run(tbl):
        def win(w, _):
            pltpu.async_copy(tbl.at[src_v.at[w]], rows_v, sem).wait()
            pltpu.sync_copy(rows_v, acc.at[dst_v.at[w]], add=True)
            return 0
        lax.fori_loop(0, WPT, win, 0)

    @pl.when(c == 0)
    def _():
        run(ta_hbm)

    @pl.when(c == 1)
    def _():
        run(tb_hbm)

    plsc.subcore_barrier()

    @pl.when(c == 0)
    def _():
        pltpu.sync_copy(acc.at[pl.ds(s * RPT, RPT)], outa_hbm.at[pl.ds(s * RPT, RPT)])

    @pl.when(c == 1)
    def _():
        pltpu.sync_copy(acc.at[pl.ds(s * RPT, RPT)], outb_hbm.at[pl.ds(s * RPT, RPT)])


@functools.partial(
    pl.kernel,
    mesh=_MESH,
    out_type=[
        jax.ShapeDtypeStruct((N_ACC, 128), jnp.float32),
        jax.ShapeDtypeStruct((N_ACC, 128), jnp.float32),
    ],
    scratch_types=[
        pltpu.VMEM((WPT, WIN), jnp.int32),
        pltpu.VMEM((WPT, WIN), jnp.int32),
        pltpu.VMEM((WIN, 128), jnp.float32),
        pltpu.VMEM_SHARED((N_ACC, 128), jnp.float32),
        pltpu.SemaphoreType.DMA,
    ],
)
def _scatter64_kernel(t_hbm, src_hbm, dst_hbm, outa_hbm, outb_hbm,
                      src_v, dst_v, rows_v, acc, sem):
    c = lax.axis_index("c")
    s = lax.axis_index("s")
    _zero_rows(rows_v, WIN, 128)
    _zero_acc(rows_v, acc, s)
    pltpu.sync_copy(src_hbm.at[s], src_v)
    pltpu.sync_copy(dst_hbm.at[s], dst_v)
    plsc.subcore_barrier()

    half = WPT // NC
    def win(w, _):
        pltpu.async_copy(t_hbm.at[src_v.at[w]], rows_v, sem).wait()
        pltpu.sync_copy(rows_v, acc.at[dst_v.at[w]], add=True)
        return 0
    lax.fori_loop(c * half, (c + 1) * half, win, 0)
    plsc.subcore_barrier()

    @pl.when(c == 0)
    def _():
        pltpu.sync_copy(acc.at[pl.ds(s * RPT, RPT)], outa_hbm.at[pl.ds(s * RPT, RPT)])

    @pl.when(c == 1)
    def _():
        pltpu.sync_copy(acc.at[pl.ds(s * RPT, RPT)], outb_hbm.at[pl.ds(s * RPT, RPT)])


_BM = 1000


def _prep_body(x_ref, da_ref, db_ref, norm_ref, pa_ref, pb_ref):
    deg = da_ref[...][:, :16] + db_ref[...][:, :16]
    norm = jnp.where(deg > 0.0, lax.rsqrt(jnp.maximum(deg, 1.0)), 0.0)
    norm_ref[...] = norm
    n1 = norm[:, 0:1]
    x = x_ref[...]
    pa_ref[...] = x[:, :128] * n1
    pb_ref[...] = x[:, 128:] * n1


def _prep_call(x, dega, degb):
    return pl.pallas_call(
        _prep_body,
        grid=(N // _BM,),
        in_specs=[
            pl.BlockSpec((_BM, D_IN), lambda m: (m, 0)),
            pl.BlockSpec((_BM, 128), lambda m: (m, 0)),
            pl.BlockSpec((_BM, 128), lambda m: (m, 0)),
        ],
        out_specs=[
            pl.BlockSpec((_BM, 16), lambda m: (m, 0)),
            pl.BlockSpec((_BM, 128), lambda m: (m, 0)),
            pl.BlockSpec((_BM, 128), lambda m: (m, 0)),
        ],
        out_shape=[
            jax.ShapeDtypeStruct((N, 16), jnp.float32),
            jax.ShapeDtypeStruct((N, 128), jnp.float32),
            jax.ShapeDtypeStruct((N, 128), jnp.float32),
        ],
    )(x, dega, degb)


def _mm1_body(aa_ref, ab_ref, w_ref, b_ref, nrm_ref, oa_ref, ob_ref):
    n1 = nrm_ref[...][:, 0:1]
    h = jnp.dot(aa_ref[...], w_ref[0:128, :], preferred_element_type=jnp.float32)
    h = h + jnp.dot(ab_ref[...], w_ref[128:256, :], preferred_element_type=jnp.float32)
    h = jnp.maximum(h * n1 + b_ref[...], 0.0) * n1
    oa_ref[...] = h[:, :128]
    ob_ref[...] = h[:, 128:]


def _mm1_call(aa, ab, w, b, norm16):
    return pl.pallas_call(
        _mm1_body,
        grid=(N // _BM,),
        in_specs=[
            pl.BlockSpec((_BM, 128), lambda m: (m, 0)),
            pl.BlockSpec((_BM, 128), lambda m: (m, 0)),
            pl.BlockSpec((D_HID, D_HID), lambda m: (0, 0)),
            pl.BlockSpec((1, D_HID), lambda m: (0, 0)),
            pl.BlockSpec((_BM, 16), lambda m: (m, 0)),
        ],
        out_specs=[
            pl.BlockSpec((_BM, 128), lambda m: (m, 0)),
            pl.BlockSpec((_BM, 128), lambda m: (m, 0)),
        ],
        out_shape=[
            jax.ShapeDtypeStruct((N, 128), jnp.float32),
            jax.ShapeDtypeStruct((N, 128), jnp.float32),
        ],
    )(aa, ab, w, b, norm16)


def _mm23_body(aa_ref, ab_ref, w2_ref, b2_ref, w3_ref, nrm_ref, o_ref):
    n1 = nrm_ref[...][:, 0:1]
    h = jnp.dot(aa_ref[...], w2_ref[0:128, :], preferred_element_type=jnp.float32)
    h = h + jnp.dot(ab_ref[...], w2_ref[128:256, :], preferred_element_type=jnp.float32)
    h = jnp.maximum(h * n1 + b2_ref[...], 0.0)
    o_ref[...] = jnp.dot(h, w3_ref[...], preferred_element_type=jnp.float32) * n1


def _mm23_call(aa, ab, w2, b2, w3p, norm16):
    return pl.pallas_call(
        _mm23_body,
        grid=(N // _BM,),
        in_specs=[
            pl.BlockSpec((_BM, 128), lambda m: (m, 0)),
            pl.BlockSpec((_BM, 128), lambda m: (m, 0)),
            pl.BlockSpec((D_HID, D_HID), lambda m: (0, 0)),
            pl.BlockSpec((1, D_HID), lambda m: (0, 0)),
            pl.BlockSpec((D_HID, 128), lambda m: (0, 0)),
            pl.BlockSpec((_BM, 16), lambda m: (m, 0)),
        ],
        out_specs=[pl.BlockSpec((_BM, 128), lambda m: (m, 0))],
        out_shape=[jax.ShapeDtypeStruct((N, 128), jnp.float32)],
    )(aa, ab, w2, b2, w3p, norm16)[0]


def _fin_body(aa_ref, ab_ref, b3_ref, nrm_ref, o_ref):
    n1 = nrm_ref[...][:, 0:1]
    o_ref[...] = (aa_ref[...][:, :D_OUT] + ab_ref[...][:, :D_OUT]) * n1 + b3_ref[...]


def _fin_call(aa, ab, b3, norm16):
    return pl.pallas_call(
        _fin_body,
        grid=(N // _BM,),
        in_specs=[
            pl.BlockSpec((_BM, 128), lambda m: (m, 0)),
            pl.BlockSpec((_BM, 128), lambda m: (m, 0)),
            pl.BlockSpec((1, D_OUT), lambda m: (0, 0)),
            pl.BlockSpec((_BM, 16), lambda m: (m, 0)),
        ],
        out_specs=[pl.BlockSpec((_BM, D_OUT), lambda m: (m, 0))],
        out_shape=[jax.ShapeDtypeStruct((N, D_OUT), jnp.float32)],
    )(aa, ab, b3, norm16)[0]


def kernel(features, edge_index, W1, b1, W2, b2, W3, b3):
    src = edge_index[0]
    dst = edge_index[1]
    npad = E_PAD - E
    psrc = (jnp.arange(npad, dtype=jnp.int32) * 97) % N
    pdst = N + (jnp.arange(npad, dtype=jnp.int32) % 16)
    srcw = jnp.concatenate([src, psrc]).reshape(NS, WPT, WIN)
    dstw = jnp.concatenate([dst, pdst]).reshape(NS, WPT, WIN)

    dega, degb = _deg_kernel(dstw)
    norm16, p0a, p0b = _prep_call(features, dega, degb)
    a1a, a1b = _scatter128_kernel(p0a, p0b, srcw, dstw)
    p1a, p1b = _mm1_call(a1a, a1b, W1, b1.reshape(1, -1), norm16)
    a2a, a2b = _scatter128_kernel(p1a, p1b, srcw, dstw)
    t3 = _mm23_call(a2a, a2b, W2, b2.reshape(1, -1),
                    jnp.pad(W3, ((0, 0), (0, 128 - D_OUT))), norm16)
    a3a, a3b = _scatter64_kernel(t3, srcw, dstw)
    return _fin_call(a3a, a3b, b3.reshape(1, -1), norm16)


# speedup vs baseline: 7.6219x; 1.4372x over previous
import functools

import jax
import jax.numpy as jnp
from jax import lax
from jax.experimental import pallas as pl
from jax.experimental.pallas import tpu as pltpu
from jax.experimental.pallas import tpu_sc as plsc

N = 10000
E = 160000
D_IN = 256
D_HID = 256
D_OUT = 64

NC = 2
NS = 16
WIN = 128
WPT = 80
E_PAD = NS * WPT * WIN
N_ACC = 10112
RPT = N_ACC // NS
ZPT = N_ACC // NS

_MESH = plsc.VectorSubcoreMesh(core_axis_name="c", subcore_axis_name="s")


def _zero_rows(buf, nrows, ncols):
    def row(i, _):
        for j in range(ncols // 16):
            buf[i, pl.ds(j * 16, 16)] = jnp.zeros((16,), jnp.float32)
        return 0
    lax.fori_loop(0, nrows, row, 0)


def _zero_acc(z_buf, acc, s):
    for k in range(ZPT // WIN + 1):
        st = s * ZPT + min(k * WIN, ZPT - WIN)
        pltpu.sync_copy(z_buf, acc.at[pl.ds(st, WIN)])


GW = 8


def _zero_rows3(buf, nrows, ncols):
    def row(i, _):
        for j in range(ncols // 16):
            buf[0, i, pl.ds(j * 16, 16)] = jnp.zeros((16,), jnp.float32)
        return 0
    lax.fori_loop(0, nrows, row, 0)


def _run_pipe(tbl, acc, sidx, didx, rows, gsems, ssems, nwl):
    def gwait(rb):
        pltpu.make_async_copy(tbl.at[sidx.at[0]], rows.at[rb], gsems[rb]).wait()

    def swait(rb):
        pltpu.make_async_copy(rows.at[rb], acc.at[didx.at[0]], ssems[rb]).wait()

    pltpu.async_copy(tbl.at[sidx.at[0]], rows.at[0], gsems[0])

    def body(k, _):
        for par in range(2):
            w = 2 * k + par
            rb = par

            @pl.when(w + 1 < nwl)
            def _():
                @pl.when(w >= 1)
                def _():
                    swait(rb ^ 1)
                pltpu.async_copy(tbl.at[sidx.at[w + 1]], rows.at[rb ^ 1],
                                 gsems[rb ^ 1])

            gwait(rb)
            pltpu.async_copy(rows.at[rb], acc.at[didx.at[w]], ssems[rb],
                             add=True)
        return 0

    lax.fori_loop(0, nwl // 2, body, 0)
    swait(0) if (nwl - 2) % 2 == 0 else swait(1)
    swait((nwl - 1) % 2)


@functools.partial(
    pl.kernel,
    mesh=_MESH,
    out_type=[
        jax.ShapeDtypeStruct((N_ACC, 128), jnp.float32),
        jax.ShapeDtypeStruct((N_ACC, 128), jnp.float32),
    ],
    scratch_types=[
        pltpu.VMEM((WPT, WIN), jnp.int32),
        pltpu.VMEM((WIN, 128), jnp.float32),
        pltpu.VMEM_SHARED((N_ACC, 128), jnp.float32),
        pltpu.SemaphoreType.DMA,
    ],
)
def _deg_kernel(dst_hbm, outa_hbm, outb_hbm, dst_v, ones_v, acc, sem):
    c = lax.axis_index("c")
    s = lax.axis_index("s")

    def zrow(i, _):
        for j in range(8):
            ones_v[i, pl.ds(j * 16, 16)] = jnp.zeros((16,), jnp.float32)
        return 0
    lax.fori_loop(0, WIN, zrow, 0)
    _zero_acc(ones_v, acc, s)
    def orow(i, _):
        ones_v[i, pl.ds(0, 16)] = jnp.ones((16,), jnp.float32)
        return 0
    lax.fori_loop(0, WIN, orow, 0)
    pltpu.sync_copy(dst_hbm.at[s], dst_v)
    plsc.subcore_barrier()

    half = WPT // NC
    FIRE = 8
    def grp(k, _):
        w0 = c * half + k * FIRE
        for j in range(FIRE):
            pltpu.async_copy(ones_v, acc.at[dst_v.at[w0 + j]], sem, add=True)
        for j in range(FIRE):
            pltpu.make_async_copy(ones_v, acc.at[dst_v.at[w0 + j]], sem).wait()
        return 0
    lax.fori_loop(0, half // FIRE, grp, 0)
    plsc.subcore_barrier()

    @pl.when(c == 0)
    def _():
        pltpu.sync_copy(acc.at[pl.ds(s * RPT, RPT)], outa_hbm.at[pl.ds(s * RPT, RPT)])

    @pl.when(c == 1)
    def _():
        pltpu.sync_copy(acc.at[pl.ds(s * RPT, RPT)], outb_hbm.at[pl.ds(s * RPT, RPT)])


@functools.partial(
    pl.kernel,
    mesh=_MESH,
    out_type=[
        jax.ShapeDtypeStruct((N_ACC, 128), jnp.float32),
        jax.ShapeDtypeStruct((N_ACC, 128), jnp.float32),
    ],
    scratch_types=[
        pltpu.VMEM((2, GW, WIN), jnp.int32),
        pltpu.VMEM((2, GW, WIN), jnp.int32),
        pltpu.VMEM((2, WIN, 128), jnp.float32),
        pltpu.VMEM_SHARED((N_ACC, 128), jnp.float32),
    ] + [pltpu.SemaphoreType.DMA] * 6,
)
def _scatter128_kernel(ta_hbm, tb_hbm, src_hbm, dst_hbm, outa_hbm, outb_hbm,
                       sidx, didx, rows_v, acc, *sems):
    isems = sems[:2]
    gsems = sems[2:4]
    ssems = sems[4:6]
    c = lax.axis_index("c")
    s = lax.axis_index("s")
    _zero_rows3(rows_v, WIN, 128)
    _zero_acc(rows_v.at[0], acc, s)
    plsc.subcore_barrier()

    ngrp = WPT // GW

    def iload(g, slot):
        pltpu.async_copy(src_hbm.at[s, pl.ds(g * GW, GW)], sidx.at[slot], isems[slot])
        pltpu.async_copy(dst_hbm.at[s, pl.ds(g * GW, GW)], didx.at[slot], isems[slot])

    def iwait(slot):
        pltpu.make_async_copy(src_hbm.at[s, pl.ds(0, GW)], sidx.at[slot], isems[slot]).wait()
        pltpu.make_async_copy(dst_hbm.at[s, pl.ds(0, GW)], didx.at[slot], isems[slot]).wait()

    def run(tbl):
        def gwait(rb):
            pltpu.make_async_copy(tbl.at[sidx.at[0, 0]], rows_v.at[rb], gsems[rb]).wait()

        def swait(rb):
            pltpu.make_async_copy(rows_v.at[rb], acc.at[didx.at[0, 0]], ssems[rb]).wait()

        iload(0, 0)
        iwait(0)
        pltpu.async_copy(tbl.at[sidx.at[0, 0]], rows_v.at[0], gsems[0])

        def grp_pair(k, _):
            for par in range(2):
                g = 2 * k + par

                for j in range(GW):
                    w = g * GW + j
                    rb = j % 2
                    nslot, nj = (par, j + 1) if j + 1 < GW else (par ^ 1, 0)

                    @pl.when(w + 1 < WPT)
                    def _():
                        @pl.when(w >= 1)
                        def _():
                            swait(rb ^ 1)
                        if j == 0:
                            @pl.when(g + 1 < ngrp)
                            def _():
                                iload(g + 1, par ^ 1)
                        if nj == 0:
                            iwait(nslot)
                        pltpu.async_copy(tbl.at[sidx.at[nslot, nj]],
                                         rows_v.at[rb ^ 1], gsems[rb ^ 1])

                    gwait(rb)
                    pltpu.async_copy(rows_v.at[rb], acc.at[didx.at[par, j]],
                                     ssems[rb], add=True)
            return 0

        lax.fori_loop(0, ngrp // 2, grp_pair, 0)
        swait((WPT - 2) % 2)
        swait((WPT - 1) % 2)

    @pl.when(c == 0)
    def _():
        run(ta_hbm)

    @pl.when(c == 1)
    def _():
        run(tb_hbm)

    plsc.subcore_barrier()

    @pl.when(c == 0)
    def _():
        pltpu.sync_copy(acc.at[pl.ds(s * RPT, RPT)], outa_hbm.at[pl.ds(s * RPT, RPT)])

    @pl.when(c == 1)
    def _():
        pltpu.sync_copy(acc.at[pl.ds(s * RPT, RPT)], outb_hbm.at[pl.ds(s * RPT, RPT)])


@functools.partial(
    pl.kernel,
    mesh=_MESH,
    out_type=[
        jax.ShapeDtypeStruct((N_ACC, 128), jnp.float32),
        jax.ShapeDtypeStruct((N_ACC, 128), jnp.float32),
    ],
    scratch_types=[
        pltpu.VMEM((WPT // NC, WIN), jnp.int32),
        pltpu.VMEM((WPT // NC, WIN), jnp.int32),
        pltpu.VMEM((2, WIN, 128), jnp.float32),
        pltpu.VMEM_SHARED((N_ACC, 128), jnp.float32),
    ] + [pltpu.SemaphoreType.DMA] * 4,
)
def _scatter64_kernel(t_hbm, src_hbm, dst_hbm, outa_hbm, outb_hbm,
                      src_v, dst_v, rows_v, acc, *sems):
    gsems = sems[:2]
    ssems = sems[2:4]
    c = lax.axis_index("c")
    s = lax.axis_index("s")
    half = WPT // NC
    _zero_rows3(rows_v, WIN, 128)
    _zero_acc(rows_v.at[0], acc, s)
    pltpu.sync_copy(src_hbm.at[s, pl.ds(c * half, half)], src_v)
    pltpu.sync_copy(dst_hbm.at[s, pl.ds(c * half, half)], dst_v)
    plsc.subcore_barrier()

    _run_pipe(t_hbm, acc, src_v, dst_v, rows_v, gsems, ssems, half)
    plsc.subcore_barrier()

    @pl.when(c == 0)
    def _():
        pltpu.sync_copy(acc.at[pl.ds(s * RPT, RPT)], outa_hbm.at[pl.ds(s * RPT, RPT)])

    @pl.when(c == 1)
    def _():
        pltpu.sync_copy(acc.at[pl.ds(s * RPT, RPT)], outb_hbm.at[pl.ds(s * RPT, RPT)])


_BM = 1000


def _prep_body(x_ref, da_ref, db_ref, norm_ref, pa_ref, pb_ref):
    deg = da_ref[...][:, :16] + db_ref[...][:, :16]
    norm = jnp.where(deg > 0.0, lax.rsqrt(jnp.maximum(deg, 1.0)), 0.0)
    norm_ref[...] = norm
    n1 = norm[:, 0:1]
    x = x_ref[...]
    pa_ref[...] = x[:, :128] * n1
    pb_ref[...] = x[:, 128:] * n1


def _prep_call(x, dega, degb):
    return pl.pallas_call(
        _prep_body,
        grid=(N // _BM,),
        in_specs=[
            pl.BlockSpec((_BM, D_IN), lambda m: (m, 0)),
            pl.BlockSpec((_BM, 128), lambda m: (m, 0)),
            pl.BlockSpec((_BM, 128), lambda m: (m, 0)),
        ],
        out_specs=[
            pl.BlockSpec((_BM, 16), lambda m: (m, 0)),
            pl.BlockSpec((_BM, 128), lambda m: (m, 0)),
            pl.BlockSpec((_BM, 128), lambda m: (m, 0)),
        ],
        out_shape=[
            jax.ShapeDtypeStruct((N, 16), jnp.float32),
            jax.ShapeDtypeStruct((N, 128), jnp.float32),
            jax.ShapeDtypeStruct((N, 128), jnp.float32),
        ],
    )(x, dega, degb)


def _mm1_body(aa_ref, ab_ref, w_ref, b_ref, nrm_ref, oa_ref, ob_ref):
    n1 = nrm_ref[...][:, 0:1]
    h = jnp.dot(aa_ref[...], w_ref[0:128, :], preferred_element_type=jnp.float32)
    h = h + jnp.dot(ab_ref[...], w_ref[128:256, :], preferred_element_type=jnp.float32)
    h = jnp.maximum(h * n1 + b_ref[...], 0.0) * n1
    oa_ref[...] = h[:, :128]
    ob_ref[...] = h[:, 128:]


def _mm1_call(aa, ab, w, b, norm16):
    return pl.pallas_call(
        _mm1_body,
        grid=(N // _BM,),
        in_specs=[
            pl.BlockSpec((_BM, 128), lambda m: (m, 0)),
            pl.BlockSpec((_BM, 128), lambda m: (m, 0)),
            pl.BlockSpec((D_HID, D_HID), lambda m: (0, 0)),
            pl.BlockSpec((1, D_HID), lambda m: (0, 0)),
            pl.BlockSpec((_BM, 16), lambda m: (m, 0)),
        ],
        out_specs=[
            pl.BlockSpec((_BM, 128), lambda m: (m, 0)),
            pl.BlockSpec((_BM, 128), lambda m: (m, 0)),
        ],
        out_shape=[
            jax.ShapeDtypeStruct((N, 128), jnp.float32),
            jax.ShapeDtypeStruct((N, 128), jnp.float32),
        ],
    )(aa, ab, w, b, norm16)


def _mm23_body(aa_ref, ab_ref, w2_ref, b2_ref, w3_ref, nrm_ref, o_ref):
    n1 = nrm_ref[...][:, 0:1]
    h = jnp.dot(aa_ref[...], w2_ref[0:128, :], preferred_element_type=jnp.float32)
    h = h + jnp.dot(ab_ref[...], w2_ref[128:256, :], preferred_element_type=jnp.float32)
    h = jnp.maximum(h * n1 + b2_ref[...], 0.0)
    o_ref[...] = jnp.dot(h, w3_ref[...], preferred_element_type=jnp.float32) * n1


def _mm23_call(aa, ab, w2, b2, w3p, norm16):
    return pl.pallas_call(
        _mm23_body,
        grid=(N // _BM,),
        in_specs=[
            pl.BlockSpec((_BM, 128), lambda m: (m, 0)),
            pl.BlockSpec((_BM, 128), lambda m: (m, 0)),
            pl.BlockSpec((D_HID, D_HID), lambda m: (0, 0)),
            pl.BlockSpec((1, D_HID), lambda m: (0, 0)),
            pl.BlockSpec((D_HID, 128), lambda m: (0, 0)),
            pl.BlockSpec((_BM, 16), lambda m: (m, 0)),
        ],
        out_specs=[pl.BlockSpec((_BM, 128), lambda m: (m, 0))],
        out_shape=[jax.ShapeDtypeStruct((N, 128), jnp.float32)],
    )(aa, ab, w2, b2, w3p, norm16)[0]


def _fin_body(aa_ref, ab_ref, b3_ref, nrm_ref, o_ref):
    n1 = nrm_ref[...][:, 0:1]
    o_ref[...] = (aa_ref[...][:, :D_OUT] + ab_ref[...][:, :D_OUT]) * n1 + b3_ref[...]


def _fin_call(aa, ab, b3, norm16):
    return pl.pallas_call(
        _fin_body,
        grid=(N // _BM,),
        in_specs=[
            pl.BlockSpec((_BM, 128), lambda m: (m, 0)),
            pl.BlockSpec((_BM, 128), lambda m: (m, 0)),
            pl.BlockSpec((1, D_OUT), lambda m: (0, 0)),
            pl.BlockSpec((_BM, 16), lambda m: (m, 0)),
        ],
        out_specs=[pl.BlockSpec((_BM, D_OUT), lambda m: (m, 0))],
        out_shape=[jax.ShapeDtypeStruct((N, D_OUT), jnp.float32)],
    )(aa, ab, b3, norm16)[0]


def kernel(features, edge_index, W1, b1, W2, b2, W3, b3):
    src = edge_index[0]
    dst = edge_index[1]
    npad = E_PAD - E
    psrc = (jnp.arange(npad, dtype=jnp.int32) * 97) % N
    pdst = N + (jnp.arange(npad, dtype=jnp.int32) % 16)
    srcw = jnp.concatenate([src, psrc]).reshape(NS, WPT, WIN)
    dstw = jnp.concatenate([dst, pdst]).reshape(NS, WPT, WIN)

    dega, degb = _deg_kernel(dstw)
    norm16, p0a, p0b = _prep_call(features, dega, degb)
    a1a, a1b = _scatter128_kernel(p0a, p0b, srcw, dstw)
    p1a, p1b = _mm1_call(a1a, a1b, W1, b1.reshape(1, -1), norm16)
    a2a, a2b = _scatter128_kernel(p1a, p1b, srcw, dstw)
    t3 = _mm23_call(a2a, a2b, W2, b2.reshape(1, -1),
                    jnp.pad(W3, ((0, 0), (0, 128 - D_OUT))), norm16)
    a3a, a3b = _scatter64_kernel(t3, srcw, dstw)
    return _fin_call(a3a, a3b, b3.reshape(1, -1), norm16)
